# baseline (speedup 1.0000x reference)
import functools
import math

import jax
import jax.numpy as jnp
import numpy as np
from jax import lax
from jax.experimental import pallas as pl
from jax.experimental.pallas import tpu as pltpu
from jax.experimental.pallas import tpu_sc as plsc

D_MODEL = 1024
MAX_SEQ_LEN = 2048
_SCALE = math.sqrt(D_MODEL)

_NC, _NS, _L = 2, 16, 16
_NW = _NC * _NS
_CP = 8
_NSB = 3


def _sinusoidal_pe(max_seq_len: int, d_model: int) -> np.ndarray:
    position = np.arange(0, max_seq_len, dtype=np.float32)[:, None]
    div_term = np.exp(
        np.arange(0, d_model, 2, dtype=np.float32)
        * np.float32(-math.log(10000.0) / d_model)
    ).astype(np.float32)
    pe = np.zeros((max_seq_len, d_model), dtype=np.float32)
    pe[:, 0::2] = np.sin(position * div_term, dtype=np.float32)
    pe[:, 1::2] = np.cos(position * div_term, dtype=np.float32)
    return pe


def _embed(xf, pe, table, *, b_dim, s):
    ppw = s // _NW
    nsc = ppw // _CP
    rows = b_dim * _CP
    mesh = plsc.VectorSubcoreMesh(core_axis_name="c", subcore_axis_name="s")

    @functools.partial(
        pl.kernel,
        out_type=jax.ShapeDtypeStruct((b_dim * s, D_MODEL), jnp.float32),
        mesh=mesh,
        scratch_types=[
            pltpu.VMEM((b_dim, ppw), jnp.int32),
        ]
        + [pltpu.VMEM((rows, D_MODEL), jnp.float32) for _ in range(_NSB)]
        + [pltpu.VMEM((_CP, D_MODEL), jnp.float32) for _ in range(_NSB)]
        + [pltpu.SemaphoreType.DMA for _ in range(3 * _NSB)],
    )
    def k(xf_hbm, pe_hbm, table_hbm, out_hbm, idx_v, *rest):
        bufs = rest[:_NSB]
        pe_bufs = rest[_NSB : 2 * _NSB]
        g_sems = rest[2 * _NSB : 3 * _NSB]
        o_sems = rest[3 * _NSB : 4 * _NSB]
        pe_sems = rest[4 * _NSB :]
        wid = lax.axis_index("s") * _NC + lax.axis_index("c")
        pbase = wid * ppw

        for b in range(b_dim):
            pltpu.sync_copy(xf_hbm.at[pl.ds(b * s + pbase, ppw)], idx_v.at[b])

        def issue_pe(u):
            return pltpu.async_copy(
                pe_hbm.at[pl.ds(pbase + u * _CP, _CP), :],
                pe_bufs[u % _NSB],
                pe_sems[u % _NSB],
            )

        def issue_gathers(u):
            nb = u % _NSB
            return [
                pltpu.async_copy(
                    table_hbm.at[idx_v.at[b, pl.ds(u * _CP, _CP)]],
                    bufs[nb].at[pl.ds(b * _CP, _CP), :],
                    g_sems[nb],
                )
                for b in range(b_dim)
            ]

        pe_dma = [None] * _NSB
        g_dma = [None] * _NSB
        out_dma = [None] * _NSB
        for u in range(min(2, nsc)):
            g_dma[u % _NSB] = issue_gathers(u)
            pe_dma[u % _NSB] = issue_pe(u)

        for u in range(nsc):
            nb = u % _NSB
            buf = bufs[nb]
            pe_v = pe_bufs[nb]
            if u + 2 < nsc:
                nb2 = (u + 2) % _NSB
                if out_dma[nb2] is not None:
                    for d in out_dma[nb2]:
                        d.wait()
                    out_dma[nb2] = None
                g_dma[nb2] = issue_gathers(u + 2)
                pe_dma[nb2] = issue_pe(u + 2)
            pe_dma[nb].wait()
            for d in g_dma[nb]:
                d.wait()

            @plsc.parallel_loop(0, _CP * (D_MODEL // _L), 1, unroll=4)
            def _fma(kk):
                i = lax.shift_right_logical(kk, 6)
                j = pl.multiple_of(
                    lax.shift_left(lax.bitwise_and(kk, D_MODEL // _L - 1), 4), _L
                )
                p = pe_v[i, pl.ds(j, _L)]
                for b in range(b_dim):
                    r = b * _CP + i
                    buf[r, pl.ds(j, _L)] = buf[r, pl.ds(j, _L)] * _SCALE + p

            out_dma[nb] = [
                pltpu.async_copy(
                    buf.at[pl.ds(b * _CP, _CP), :],
                    out_hbm.at[pl.ds(b * s + pbase + u * _CP, _CP), :],
                    o_sems[nb],
                )
                for b in range(b_dim)
            ]
        for nb in range(_NSB):
            if out_dma[nb] is not None:
                for d in out_dma[nb]:
                    d.wait()

    return k(xf, pe, table)


def kernel(x, table):
    b_dim, s = x.shape
    pe = _sinusoidal_pe(MAX_SEQ_LEN, D_MODEL)[:s]
    xf = x.reshape(b_dim * s).astype(jnp.int32)
    out = _embed(xf, pe, table, b_dim=b_dim, s=s)
    return out.reshape(b_dim, s, D_MODEL)


# speedup vs baseline: 1.1208x; 1.1208x over previous
import functools
import math

import jax
import jax.numpy as jnp
import numpy as np
from jax import lax
from jax.experimental import pallas as pl
from jax.experimental.pallas import tpu as pltpu
from jax.experimental.pallas import tpu_sc as plsc

D_MODEL = 1024
MAX_SEQ_LEN = 2048
_SCALE = math.sqrt(D_MODEL)

_NC, _NS, _L = 2, 16, 16
_NW = _NC * _NS
_CP = 16
_NBUF = 5
_AHEAD = 3
_DW = D_MODEL // 2


def _sinusoidal_pe(max_seq_len: int, d_model: int) -> np.ndarray:
    position = np.arange(0, max_seq_len, dtype=np.float32)[:, None]
    div_term = np.exp(
        np.arange(0, d_model, 2, dtype=np.float32)
        * np.float32(-math.log(10000.0) / d_model)
    ).astype(np.float32)
    pe = np.zeros((max_seq_len, d_model), dtype=np.float32)
    pe[:, 0::2] = np.sin(position * div_term, dtype=np.float32)
    pe[:, 1::2] = np.cos(position * div_term, dtype=np.float32)
    return pe


def _bf16_bits(f: np.ndarray) -> np.ndarray:
    u = f.astype(np.float32).view(np.uint32).astype(np.uint64)
    u = u + 0x7FFF + ((u >> 16) & 1)
    return ((u >> 16) & 0xFFFF).astype(np.uint32)


def _packed_pe(s: int) -> np.ndarray:
    pe = _sinusoidal_pe(MAX_SEQ_LEN, D_MODEL)[:s]
    blocks = pe.reshape(s, D_MODEL // 32, 2, 16)
    hi = _bf16_bits(blocks[:, :, 0, :])
    lo = _bf16_bits(blocks[:, :, 1, :])
    return ((hi << 16) | lo).reshape(s, _DW)


def _embed(xf, pe_packed, table, *, b_dim, s):
    ppw = s // _NW
    pcb = ppw // _CP
    nchunk = b_dim * pcb
    mesh = plsc.VectorSubcoreMesh(core_axis_name="c", subcore_axis_name="s")

    @functools.partial(
        pl.kernel,
        out_type=jax.ShapeDtypeStruct((b_dim * s, D_MODEL), jnp.float32),
        mesh=mesh,
        scratch_types=[
            pltpu.VMEM((b_dim, ppw), jnp.int32),
            pltpu.VMEM((_CP, _DW), jnp.uint32),
            pltpu.VMEM((_CP, _DW), jnp.uint32),
        ]
        + [pltpu.VMEM((_CP, D_MODEL), jnp.float32) for _ in range(_NBUF)]
        + [pltpu.SemaphoreType.DMA for _ in range(2 + 2 * _NBUF)],
    )
    def k(xf_hbm, pe_hbm, table_hbm, out_hbm, idx_v, pe_v0, pe_v1, *rest):
        bufs = rest[:_NBUF]
        pe_sems = rest[_NBUF : _NBUF + 2]
        g_sems = rest[_NBUF + 2 : 2 * _NBUF + 2]
        o_sems = rest[2 * _NBUF + 2 :]
        pe_bufs = (pe_v0, pe_v1)
        wid = lax.axis_index("s") * _NC + lax.axis_index("c")
        pbase = wid * ppw

        for b in range(b_dim):
            pltpu.sync_copy(xf_hbm.at[pl.ds(b * s + pbase, ppw)], idx_v.at[b])

        def issue_pe(o):
            return pltpu.async_copy(
                pe_hbm.at[pl.ds(pbase + o * _CP, _CP), :],
                pe_bufs[o % 2],
                pe_sems[o % 2],
            )

        def issue_gather(c):
            o, b = divmod(c, b_dim)
            return pltpu.async_copy(
                table_hbm.at[idx_v.at[b, pl.ds(o * _CP, _CP)]],
                bufs[c % _NBUF],
                g_sems[c % _NBUF],
            )

        pe_dma = [None, None]
        pe_dma[0] = issue_pe(0)
        if pcb > 1:
            pe_dma[1] = issue_pe(1)

        g_dma = [None] * _NBUF
        out_dma = [None] * _NBUF
        for c in range(min(_AHEAD, nchunk)):
            g_dma[c % _NBUF] = issue_gather(c)

        for c in range(nchunk):
            o, b = divmod(c, b_dim)
            nb = c % _NBUF
            buf = bufs[nb]
            if b == 0:
                pe_dma[o % 2].wait()
            pe_v = pe_bufs[o % 2]
            g_dma[nb].wait()
            if c + _AHEAD < nchunk:
                nb2 = (c + _AHEAD) % _NBUF
                if out_dma[nb2] is not None:
                    out_dma[nb2].wait()
                g_dma[nb2] = issue_gather(c + _AHEAD)

            @plsc.parallel_loop(0, _CP * (D_MODEL // 32), 1, unroll=8)
            def _fma(kk):
                i = lax.shift_right_logical(kk, 5)
                kb = lax.bitwise_and(kk, D_MODEL // 32 - 1)
                jj = pl.multiple_of(lax.shift_left(kb, 4), _L)
                j = pl.multiple_of(lax.shift_left(kb, 5), 32)
                pv = pe_v[i, pl.ds(jj, _L)]
                pa = lax.bitcast_convert_type(
                    lax.bitwise_and(pv, jnp.uint32(0xFFFF0000)), jnp.float32
                )
                pb = lax.bitcast_convert_type(
                    lax.shift_left(pv, jnp.uint32(16)), jnp.float32
                )
                buf[i, pl.ds(j, _L)] = buf[i, pl.ds(j, _L)] * _SCALE + pa
                buf[i, pl.ds(j + _L, _L)] = (
                    buf[i, pl.ds(j + _L, _L)] * _SCALE + pb
                )

            out_dma[nb] = pltpu.async_copy(
                buf, out_hbm.at[pl.ds(b * s + pbase + o * _CP, _CP), :], o_sems[nb]
            )
            if b == b_dim - 1 and o + 2 < pcb:
                pe_dma[o % 2] = issue_pe(o + 2)
        for nb in range(_NBUF):
            if out_dma[nb] is not None:
                out_dma[nb].wait()

    return k(xf, pe_packed, table)


def kernel(x, table):
    b_dim, s = x.shape
    pe_packed = _packed_pe(s)
    xf = x.reshape(b_dim * s).astype(jnp.int32)
    out = _embed(xf, pe_packed, table, b_dim=b_dim, s=s)
    return out.reshape(b_dim, s, D_MODEL)


# speedup vs baseline: 1.1459x; 1.0223x over previous
import functools
import math

import jax
import jax.numpy as jnp
import numpy as np
from jax import lax
from jax.experimental import pallas as pl
from jax.experimental.pallas import tpu as pltpu
from jax.experimental.pallas import tpu_sc as plsc

D_MODEL = 1024
MAX_SEQ_LEN = 2048
_SCALE = math.sqrt(D_MODEL)

_NC, _NS, _L = 2, 16, 16
_NW = _NC * _NS
_CP = 16
_NBUF = 5
_AHEAD = 3
_DW = D_MODEL // 2


def _sinusoidal_pe(max_seq_len: int, d_model: int) -> np.ndarray:
    position = np.arange(0, max_seq_len, dtype=np.float32)[:, None]
    div_term = np.exp(
        np.arange(0, d_model, 2, dtype=np.float32)
        * np.float32(-math.log(10000.0) / d_model)
    ).astype(np.float32)
    pe = np.zeros((max_seq_len, d_model), dtype=np.float32)
    pe[:, 0::2] = np.sin(position * div_term, dtype=np.float32)
    pe[:, 1::2] = np.cos(position * div_term, dtype=np.float32)
    return pe


def _bf16_bits(f: np.ndarray) -> np.ndarray:
    u = f.astype(np.float32).view(np.uint32).astype(np.uint64)
    u = u + 0x7FFF + ((u >> 16) & 1)
    return ((u >> 16) & 0xFFFF).astype(np.uint32)


def _packed_pe(s: int) -> np.ndarray:
    pe = _sinusoidal_pe(MAX_SEQ_LEN, D_MODEL)[:s]
    blocks = pe.reshape(s, D_MODEL // 32, 2, 16)
    hi = _bf16_bits(blocks[:, :, 0, :])
    lo = _bf16_bits(blocks[:, :, 1, :])
    return ((hi << 16) | lo).reshape(s, _DW)


def _embed(xf, pe_packed, table, *, b_dim, s):
    ppw = s // _NW
    pcb = ppw // _CP
    nchunk = b_dim * pcb
    mesh = plsc.VectorSubcoreMesh(core_axis_name="c", subcore_axis_name="s")

    @functools.partial(
        pl.kernel,
        out_type=jax.ShapeDtypeStruct((b_dim * s, D_MODEL), jnp.float32),
        mesh=mesh,
        scratch_types=[
            pltpu.VMEM((b_dim, ppw), jnp.int32),
            pltpu.VMEM((_CP, _DW), jnp.uint32),
            pltpu.VMEM((_CP, _DW), jnp.uint32),
        ]
        + [pltpu.VMEM((_CP, D_MODEL), jnp.float32) for _ in range(_NBUF)]
        + [pltpu.SemaphoreType.DMA for _ in range(2 + 2 * _NBUF)],
    )
    def k(xf_hbm, pe_hbm, table_hbm, out_hbm, idx_v, pe_v0, pe_v1, *rest):
        bufs = rest[:_NBUF]
        pe_sems = rest[_NBUF : _NBUF + 2]
        g_sems = rest[_NBUF + 2 : 2 * _NBUF + 2]
        o_sems = rest[2 * _NBUF + 2 :]
        pe_bufs = (pe_v0, pe_v1)
        wid = lax.axis_index("s") * _NC + lax.axis_index("c")
        pbase = wid * ppw

        for b in range(b_dim):
            pltpu.sync_copy(xf_hbm.at[b, pl.ds(pbase, ppw)], idx_v.at[b])

        def issue_pe(o):
            return pltpu.async_copy(
                pe_hbm.at[pl.ds(pbase + o * _CP, _CP), :],
                pe_bufs[o % 2],
                pe_sems[o % 2],
            )

        def issue_gather(c):
            o, b = divmod(c, b_dim)
            return pltpu.async_copy(
                table_hbm.at[idx_v.at[b, pl.ds(o * _CP, _CP)]],
                bufs[c % _NBUF],
                g_sems[c % _NBUF],
            )

        pe_dma = [None, None]
        pe_dma[0] = issue_pe(0)
        if pcb > 1:
            pe_dma[1] = issue_pe(1)

        g_dma = [None] * _NBUF
        out_dma = [None] * _NBUF
        for c in range(min(_AHEAD, nchunk)):
            g_dma[c % _NBUF] = issue_gather(c)

        for c in range(nchunk):
            o, b = divmod(c, b_dim)
            nb = c % _NBUF
            buf = bufs[nb]
            if b == 0:
                pe_dma[o % 2].wait()
            pe_v = pe_bufs[o % 2]
            g_dma[nb].wait()
            if c + _AHEAD < nchunk:
                nb2 = (c + _AHEAD) % _NBUF
                if out_dma[nb2] is not None:
                    out_dma[nb2].wait()
                g_dma[nb2] = issue_gather(c + _AHEAD)

            @plsc.parallel_loop(0, _CP * (D_MODEL // 32), 1, unroll=8)
            def _fma(kk):
                i = lax.shift_right_logical(kk, 5)
                kb = lax.bitwise_and(kk, D_MODEL // 32 - 1)
                jj = pl.multiple_of(lax.shift_left(kb, 4), _L)
                j = pl.multiple_of(lax.shift_left(kb, 5), 32)
                pv = pe_v[i, pl.ds(jj, _L)]
                pa = lax.bitcast_convert_type(
                    lax.bitwise_and(pv, jnp.uint32(0xFFFF0000)), jnp.float32
                )
                pb = lax.bitcast_convert_type(
                    lax.shift_left(pv, jnp.uint32(16)), jnp.float32
                )
                buf[i, pl.ds(j, _L)] = buf[i, pl.ds(j, _L)] * _SCALE + pa
                buf[i, pl.ds(j + _L, _L)] = (
                    buf[i, pl.ds(j + _L, _L)] * _SCALE + pb
                )

            out_dma[nb] = pltpu.async_copy(
                buf, out_hbm.at[pl.ds(b * s + pbase + o * _CP, _CP), :], o_sems[nb]
            )
            if b == b_dim - 1 and o + 2 < pcb:
                pe_dma[o % 2] = issue_pe(o + 2)
        for nb in range(_NBUF):
            if out_dma[nb] is not None:
                out_dma[nb].wait()

    return k(xf, pe_packed, table)


def kernel(x, table):
    b_dim, s = x.shape
    pe_packed = _packed_pe(s)
    out = _embed(x.astype(jnp.int32), pe_packed, table, b_dim=b_dim, s=s)
    return out.reshape(b_dim, s, D_MODEL)


# speedup vs baseline: 1.1466x; 1.0006x over previous
import functools
import math

import jax
import jax.numpy as jnp
import numpy as np
from jax import lax
from jax.experimental import pallas as pl
from jax.experimental.pallas import tpu as pltpu
from jax.experimental.pallas import tpu_sc as plsc

D_MODEL = 1024
MAX_SEQ_LEN = 2048
_SCALE = math.sqrt(D_MODEL)

_NC, _NS, _L = 2, 16, 16
_NW = _NC * _NS
_CP = 16
_NBUF = 5
_AHEAD = 3
_DW = D_MODEL // 2


def _sinusoidal_pe(max_seq_len: int, d_model: int) -> np.ndarray:
    position = np.arange(0, max_seq_len, dtype=np.float32)[:, None]
    div_term = np.exp(
        np.arange(0, d_model, 2, dtype=np.float32)
        * np.float32(-math.log(10000.0) / d_model)
    ).astype(np.float32)
    pe = np.zeros((max_seq_len, d_model), dtype=np.float32)
    pe[:, 0::2] = np.sin(position * div_term, dtype=np.float32)
    pe[:, 1::2] = np.cos(position * div_term, dtype=np.float32)
    return pe


def _bf16_bits(f: np.ndarray) -> np.ndarray:
    u = f.astype(np.float32).view(np.uint32).astype(np.uint64)
    u = u + 0x7FFF + ((u >> 16) & 1)
    return ((u >> 16) & 0xFFFF).astype(np.uint32)


def _packed_pe(s: int) -> np.ndarray:
    pe = _sinusoidal_pe(MAX_SEQ_LEN, D_MODEL)[:s]
    blocks = pe.reshape(s, D_MODEL // 32, 2, 16)
    hi = _bf16_bits(blocks[:, :, 0, :])
    lo = _bf16_bits(blocks[:, :, 1, :])
    return ((hi << 16) | lo).reshape(s, _DW)


def _embed(xf, pe_packed, table, *, b_dim, s):
    ppw = s // _NW
    pcb = ppw // _CP
    nchunk = b_dim * pcb
    mesh = plsc.VectorSubcoreMesh(core_axis_name="c", subcore_axis_name="s")

    @functools.partial(
        pl.kernel,
        out_type=jax.ShapeDtypeStruct((b_dim * s, D_MODEL), jnp.float32),
        mesh=mesh,
        scratch_types=[
            pltpu.VMEM((b_dim, ppw), jnp.int32),
            pltpu.VMEM((_CP, _DW), jnp.uint32),
            pltpu.VMEM((_CP, _DW), jnp.uint32),
        ]
        + [pltpu.VMEM((_CP, D_MODEL), jnp.float32) for _ in range(_NBUF)]
        + [pltpu.SemaphoreType.DMA for _ in range(2 + 2 * _NBUF)],
    )
    def k(xf_hbm, pe_hbm, table_hbm, out_hbm, idx_v, pe_v0, pe_v1, *rest):
        bufs = rest[:_NBUF]
        pe_sems = rest[_NBUF : _NBUF + 2]
        g_sems = rest[_NBUF + 2 : 2 * _NBUF + 2]
        o_sems = rest[2 * _NBUF + 2 :]
        pe_bufs = (pe_v0, pe_v1)
        wid = lax.axis_index("s") * _NC + lax.axis_index("c")
        pbase = wid * ppw

        for b in range(b_dim):
            pltpu.sync_copy(xf_hbm.at[b, pl.ds(pbase, ppw)], idx_v.at[b])

        def issue_pe(o):
            return pltpu.async_copy(
                pe_hbm.at[pl.ds(pbase + o * _CP, _CP), :],
                pe_bufs[o % 2],
                pe_sems[o % 2],
            )

        def issue_gather(c):
            o, b = divmod(c, b_dim)
            return pltpu.async_copy(
                table_hbm.at[idx_v.at[b, pl.ds(o * _CP, _CP)]],
                bufs[c % _NBUF],
                g_sems[c % _NBUF],
            )

        pe_dma = [None, None]
        pe_dma[0] = issue_pe(0)
        if pcb > 1:
            pe_dma[1] = issue_pe(1)

        g_dma = [None] * _NBUF
        out_dma = [None] * _NBUF
        for c in range(min(_AHEAD, nchunk)):
            g_dma[c % _NBUF] = issue_gather(c)

        for c in range(nchunk):
            o, b = divmod(c, b_dim)
            nb = c % _NBUF
            buf = bufs[nb]
            if b == 0:
                pe_dma[o % 2].wait()
            pe_v = pe_bufs[o % 2]
            g_dma[nb].wait()
            if c + _AHEAD < nchunk:
                nb2 = (c + _AHEAD) % _NBUF
                if out_dma[nb2] is not None:
                    out_dma[nb2].wait()
                g_dma[nb2] = issue_gather(c + _AHEAD)

            @plsc.parallel_loop(0, _CP * (D_MODEL // 32), 1, unroll=8)
            def _fma(kk):
                i = lax.shift_right_logical(kk, 5)
                kb = lax.bitwise_and(kk, D_MODEL // 32 - 1)
                jj = pl.multiple_of(lax.shift_left(kb, 4), _L)
                j = pl.multiple_of(lax.shift_left(kb, 5), 32)
                pv = pe_v[i, pl.ds(jj, _L)]
                pa = lax.bitcast_convert_type(
                    lax.bitwise_and(pv, jnp.uint32(0xFFFF0000)), jnp.float32
                )
                pb = lax.bitcast_convert_type(
                    lax.shift_left(pv, jnp.uint32(16)), jnp.float32
                )
                buf[i, pl.ds(j, _L)] = buf[i, pl.ds(j, _L)] * _SCALE + pa
                buf[i, pl.ds(j + _L, _L)] = (
                    buf[i, pl.ds(j + _L, _L)] * _SCALE + pb
                )

            out_dma[nb] = pltpu.async_copy(
                buf, out_hbm.at[pl.ds(b * s + pbase + o * _CP, _CP), :], o_sems[nb]
            )
            if b == b_dim - 1 and o + 2 < pcb:
                pe_dma[o % 2] = issue_pe(o + 2)
        for nb in range(_NBUF):
            if out_dma[nb] is not None:
                out_dma[nb].wait()

    return k(xf, pe_packed, table)


_PE_PACKED = jnp.asarray(_packed_pe(MAX_SEQ_LEN))


def kernel(x, table):
    b_dim, s = x.shape
    pe_packed = _PE_PACKED if s == MAX_SEQ_LEN else _PE_PACKED[:s]
    out = _embed(x.astype(jnp.int32), pe_packed, table, b_dim=b_dim, s=s)
    return out.reshape(b_dim, s, D_MODEL)
